# baseline (speedup 1.0000x reference)
import functools

import jax
import jax.numpy as jnp
from jax import lax
from jax.experimental import pallas as pl
from jax.experimental.pallas import tpu as pltpu
from jax.experimental.pallas import tpu_sc as plsc

_A = 32 * 1024
_M = 128 * 512
_S = 32768
_TOTAL = _A + _M + _S
_MH = _M // 2


def _body(x_hbm, a_hbm, m_hbm, s_hbm):
    cid = lax.axis_index("c")

    @pl.when(cid == 0)
    def _():
        pltpu.sync_copy(x_hbm.at[pl.ds(0, _A)], a_hbm)
        pltpu.sync_copy(x_hbm.at[pl.ds(_A, _MH)], m_hbm.at[pl.ds(0, _MH)])

    @pl.when(cid == 1)
    def _():
        pltpu.sync_copy(x_hbm.at[pl.ds(_A + _MH, _MH)], m_hbm.at[pl.ds(_MH, _MH)])
        pltpu.sync_copy(x_hbm.at[pl.ds(_A + _M, _S)], s_hbm)


_scatter = functools.partial(
    pl.kernel,
    out_type=[
        jax.ShapeDtypeStruct((_A,), jnp.float32),
        jax.ShapeDtypeStruct((_M,), jnp.float32),
        jax.ShapeDtypeStruct((_S,), jnp.float32),
    ],
    mesh=plsc.ScalarSubcoreMesh(axis_name="c", num_cores=2),
)(_body)


def kernel(logits):
    a, m, s = _scatter(logits)
    return {
        "attack": a.reshape(32, 1024),
        "move": m.reshape(128, 512),
        "select": s.reshape(1, 32768),
    }


# speedup vs baseline: 1.6486x; 1.6486x over previous
import functools

import jax
import jax.numpy as jnp
from jax import lax
from jax.experimental import pallas as pl
from jax.experimental.pallas import tpu as pltpu
from jax.experimental.pallas import tpu_sc as plsc

_A = 32 * 1024
_M = 128 * 512
_S = 32768
_TOTAL = _A + _M + _S
_Q = _TOTAL // 4


def _body(x_hbm, a_hbm, m_hbm, s_hbm, sp0, sp1, sem0, sem1):
    cid = lax.axis_index("c")

    @pl.when(cid == 0)
    def _():
        g0 = pltpu.async_copy(x_hbm.at[pl.ds(0, _Q)], sp0, sem0)
        g1 = pltpu.async_copy(x_hbm.at[pl.ds(_Q, _Q)], sp1, sem1)
        g0.wait()
        s0 = pltpu.async_copy(sp0, a_hbm, sem0)
        g1.wait()
        s1 = pltpu.async_copy(sp1, m_hbm.at[pl.ds(0, _Q)], sem1)
        s0.wait()
        s1.wait()

    @pl.when(cid == 1)
    def _():
        g0 = pltpu.async_copy(x_hbm.at[pl.ds(2 * _Q, _Q)], sp0, sem0)
        g1 = pltpu.async_copy(x_hbm.at[pl.ds(3 * _Q, _Q)], sp1, sem1)
        g0.wait()
        s0 = pltpu.async_copy(sp0, m_hbm.at[pl.ds(_Q, _Q)], sem0)
        g1.wait()
        s1 = pltpu.async_copy(sp1, s_hbm, sem1)
        s0.wait()
        s1.wait()


_scatter = functools.partial(
    pl.kernel,
    out_type=[
        jax.ShapeDtypeStruct((_A,), jnp.float32),
        jax.ShapeDtypeStruct((_M,), jnp.float32),
        jax.ShapeDtypeStruct((_S,), jnp.float32),
    ],
    mesh=plsc.ScalarSubcoreMesh(axis_name="c", num_cores=2),
    scratch_types=[
        pltpu.VMEM_SHARED((_Q,), jnp.float32),
        pltpu.VMEM_SHARED((_Q,), jnp.float32),
        pltpu.SemaphoreType.DMA,
        pltpu.SemaphoreType.DMA,
    ],
)(_body)


def kernel(logits):
    a, m, s = _scatter(logits)
    return {
        "attack": a.reshape(32, 1024),
        "move": m.reshape(128, 512),
        "select": s.reshape(1, 32768),
    }


# speedup vs baseline: 1.7148x; 1.0402x over previous
import functools

import jax
import jax.numpy as jnp
from jax import lax
from jax.experimental import pallas as pl
from jax.experimental.pallas import tpu as pltpu
from jax.experimental.pallas import tpu_sc as plsc

_A = 32 * 1024
_M = 128 * 512
_S = 32768
_TOTAL = _A + _M + _S
_Q = _TOTAL // 4


def _body(x_hbm, a_hbm, m_hbm, s_hbm, sp0, sp1, sp2, sp3, sem0, sem1, sem2, sem3):
    g0 = pltpu.async_copy(x_hbm.at[pl.ds(0, _Q)], sp0, sem0)
    g1 = pltpu.async_copy(x_hbm.at[pl.ds(_Q, _Q)], sp1, sem1)
    g2 = pltpu.async_copy(x_hbm.at[pl.ds(2 * _Q, _Q)], sp2, sem2)
    g3 = pltpu.async_copy(x_hbm.at[pl.ds(3 * _Q, _Q)], sp3, sem3)
    g0.wait()
    s0 = pltpu.async_copy(sp0, a_hbm, sem0)
    g1.wait()
    s1 = pltpu.async_copy(sp1, m_hbm.at[pl.ds(0, _Q)], sem1)
    g2.wait()
    s2 = pltpu.async_copy(sp2, m_hbm.at[pl.ds(_Q, _Q)], sem2)
    g3.wait()
    s3 = pltpu.async_copy(sp3, s_hbm, sem3)
    s0.wait()
    s1.wait()
    s2.wait()
    s3.wait()


_scatter = functools.partial(
    pl.kernel,
    out_type=[
        jax.ShapeDtypeStruct((_A,), jnp.float32),
        jax.ShapeDtypeStruct((_M,), jnp.float32),
        jax.ShapeDtypeStruct((_S,), jnp.float32),
    ],
    mesh=plsc.ScalarSubcoreMesh(axis_name="c", num_cores=1),
    scratch_types=[
        pltpu.VMEM_SHARED((_Q,), jnp.float32),
        pltpu.VMEM_SHARED((_Q,), jnp.float32),
        pltpu.VMEM_SHARED((_Q,), jnp.float32),
        pltpu.VMEM_SHARED((_Q,), jnp.float32),
        pltpu.SemaphoreType.DMA,
        pltpu.SemaphoreType.DMA,
        pltpu.SemaphoreType.DMA,
        pltpu.SemaphoreType.DMA,
    ],
)(_body)


def kernel(logits):
    a, m, s = _scatter(logits)
    return {
        "attack": a.reshape(32, 1024),
        "move": m.reshape(128, 512),
        "select": s.reshape(1, 32768),
    }


# speedup vs baseline: 1.7159x; 1.0006x over previous
import functools

import jax
import jax.numpy as jnp
from jax import lax
from jax.experimental import pallas as pl
from jax.experimental.pallas import tpu as pltpu
from jax.experimental.pallas import tpu_sc as plsc

_A = 32 * 1024
_M = 128 * 512
_S = 32768
_TOTAL = _A + _M + _S
_NCH = 8
_C = _TOTAL // _NCH


def _body(x_hbm, a_hbm, m_hbm, s_hbm, *scr):
    bufs, sems = scr[:_NCH], scr[_NCH:]
    dests = (
        [(a_hbm, i * _C) for i in range(_A // _C)]
        + [(m_hbm, i * _C) for i in range(_M // _C)]
        + [(s_hbm, i * _C) for i in range(_S // _C)]
    )
    gathers = [
        pltpu.async_copy(x_hbm.at[pl.ds(i * _C, _C)], bufs[i], sems[i])
        for i in range(_NCH)
    ]
    scatters = []
    for i in range(_NCH):
        gathers[i].wait()
        ref, off = dests[i]
        scatters.append(pltpu.async_copy(bufs[i], ref.at[pl.ds(off, _C)], sems[i]))
    for s in scatters:
        s.wait()


_scatter = functools.partial(
    pl.kernel,
    out_type=[
        jax.ShapeDtypeStruct((_A,), jnp.float32),
        jax.ShapeDtypeStruct((_M,), jnp.float32),
        jax.ShapeDtypeStruct((_S,), jnp.float32),
    ],
    mesh=plsc.ScalarSubcoreMesh(axis_name="c", num_cores=1),
    scratch_types=(
        [pltpu.VMEM_SHARED((_C,), jnp.float32)] * _NCH
        + [pltpu.SemaphoreType.DMA] * _NCH
    ),
)(_body)


def kernel(logits):
    a, m, s = _scatter(logits)
    return {
        "attack": a.reshape(32, 1024),
        "move": m.reshape(128, 512),
        "select": s.reshape(1, 32768),
    }


# speedup vs baseline: 1.7174x; 1.0009x over previous
import functools

import jax
import jax.numpy as jnp
from jax.experimental import pallas as pl
from jax.experimental.pallas import tpu as pltpu
from jax.experimental.pallas import tpu_sc as plsc

_A = 32 * 1024
_M = 128 * 512
_S = 32768
_TOTAL = _A + _M + _S
_NCH = 8
_C = _TOTAL // _NCH


def _body(x_hbm, a_hbm, m_hbm, s_hbm, *scr):
    bufs, sems = scr[:_NCH], scr[_NCH:]
    dests = (
        [(a_hbm, i * _C) for i in range(_A // _C)]
        + [(m_hbm, i * _C) for i in range(_M // _C)]
        + [(s_hbm, i * _C) for i in range(_S // _C)]
    )
    gathers = [
        pltpu.async_copy(x_hbm.at[pl.ds(i * _C, _C)], bufs[i], sems[i])
        for i in range(_NCH)
    ]
    scatters = []
    for i in range(_NCH):
        gathers[i].wait()
        ref, off = dests[i]
        scatters.append(pltpu.async_copy(bufs[i], ref.at[pl.ds(off, _C)], sems[i]))
    for s in scatters:
        s.wait()


_scatter = functools.partial(
    pl.kernel,
    out_type=[
        jax.ShapeDtypeStruct((_A,), jnp.float32),
        jax.ShapeDtypeStruct((_M,), jnp.float32),
        jax.ShapeDtypeStruct((_S,), jnp.float32),
    ],
    mesh=plsc.ScalarSubcoreMesh(axis_name="c", num_cores=1),
    scratch_types=(
        [pltpu.VMEM_SHARED((_C,), jnp.float32)] * _NCH
        + [pltpu.SemaphoreType.DMA] * _NCH
    ),
)(_body)


def kernel(logits):
    a, m, s = _scatter(logits)
    return {
        "attack": a.reshape(32, 1024),
        "move": m.reshape(128, 512),
        "select": s.reshape(1, 32768),
    }
